# baseline (speedup 1.0000x reference)
import functools

import jax
import jax.numpy as jnp
import numpy as np
from jax.experimental import pallas as pl
from jax.experimental.pallas import tpu as pltpu
from jax.experimental.pallas import tpu_sc as plsc

_SCALE = 64.0
_MARGIN = float(np.radians(28.6))
_COS_M = float(np.cos(_MARGIN))
_SIN_M = float(np.sin(_MARGIN))

_BC = 2048
_NC = 2
_NS = 16


def _sc_gather_body(n_classes, b_per_w, lbl_hbm, tab_hbm, out_hbm,
                    lbl_v, idx_v, val_v, sem):
    wid = jax.lax.axis_index("s") * _NC + jax.lax.axis_index("c")
    base = wid * b_per_w
    pltpu.sync_copy(lbl_hbm.at[pl.ds(base, b_per_w)], lbl_v)
    for q in range(b_per_w // 16):
        sl = pl.ds(q * 16, 16)
        rowid = base + q * 16 + jax.lax.iota(jnp.int32, 16)
        idx_v[sl] = rowid * n_classes + lbl_v[sl]
    pltpu.async_copy(tab_hbm.at[idx_v], val_v, sem).wait()
    pltpu.sync_copy(val_v, out_hbm.at[pl.ds(base, b_per_w)])


def _gather_label_vals(logits, labels):
    b, n = logits.shape
    b_per_w = b // (_NC * _NS)
    tab = logits.reshape(-1)
    mesh = plsc.VectorSubcoreMesh(
        core_axis_name="c", subcore_axis_name="s",
        num_cores=_NC, num_subcores=_NS)
    return pl.kernel(
        functools.partial(_sc_gather_body, n, b_per_w),
        out_type=jax.ShapeDtypeStruct((b,), jnp.float32),
        mesh=mesh,
        scratch_types=[
            pltpu.VMEM((b_per_w,), jnp.int32),
            pltpu.VMEM((b_per_w,), jnp.int32),
            pltpu.VMEM((b_per_w,), jnp.float32),
            pltpu.SemaphoreType.DMA,
        ],
    )(labels, tab)


def _lane_tree(m):
    parts = [m[:, k * 128:(k + 1) * 128] for k in range(m.shape[1] // 128)]
    while len(parts) > 1:
        nxt = [parts[i] + parts[i + 1] for i in range(0, len(parts) - 1, 2)]
        if len(parts) % 2:
            nxt.append(parts[-1])
        parts = nxt
    return parts[0]


def _dense_body(x_ref, acc_ref, *, n_classes):
    j = pl.program_id(0)
    nb = pl.num_programs(0)
    b, bc = x_ref.shape
    e = jnp.exp(x_ref[...] * _SCALE)

    @pl.when(j == 0)
    def _():
        acc_ref[...] = jnp.zeros_like(acc_ref)

    @pl.when(j < nb - 1)
    def _():
        acc_ref[...] += _lane_tree(e)

    @pl.when(j == nb - 1)
    def _():
        cols = j * bc + jax.lax.broadcasted_iota(jnp.int32, (b, bc), 1)
        acc_ref[...] += _lane_tree(jnp.where(cols < n_classes, e, 0.0))


def _combine_body(acc_ref, v_ref, out_ref):
    s_full = jnp.sum(acc_ref[...], axis=1, keepdims=True)
    v = v_ref[...]
    s_excl = s_full - jnp.exp(v * _SCALE)
    u0 = v * _COS_M - jnp.sqrt(jnp.maximum(1.0 - v * v, 0.0)) * _SIN_M
    bits = jax.lax.bitcast_convert_type(u0, jnp.int32)
    rnd = bits + 0x0FFF + jnp.bitwise_and(jax.lax.shift_right_logical(bits, 13), 1)
    rnd = jnp.bitwise_and(rnd, jnp.int32(~0x1FFF))
    u = jax.lax.bitcast_convert_type(rnd, jnp.float32)
    t = u * _SCALE
    logz = jnp.log(s_excl + jnp.exp(t))
    out_ref[0, 0] = jnp.mean(logz - t)


def kernel(logits, labels):
    b, n = logits.shape
    v = _gather_label_vals(logits, labels.astype(jnp.int32))
    nb = pl.cdiv(n, _BC)
    acc = pl.pallas_call(
        functools.partial(_dense_body, n_classes=n),
        grid=(nb,),
        in_specs=[pl.BlockSpec((b, _BC), lambda j: (0, j))],
        out_specs=pl.BlockSpec((b, 128), lambda j: (0, 0)),
        out_shape=jax.ShapeDtypeStruct((b, 128), jnp.float32),
    )(logits)
    loss = pl.pallas_call(
        _combine_body,
        out_specs=pl.BlockSpec(memory_space=pltpu.SMEM),
        out_shape=jax.ShapeDtypeStruct((1, 1), jnp.float32),
    )(acc, v.reshape(b, 1))
    return loss.reshape(())


# speedup vs baseline: 1.7376x; 1.7376x over previous
import functools

import jax
import jax.numpy as jnp
import numpy as np
from jax.experimental import pallas as pl
from jax.experimental.pallas import tpu as pltpu

_SCALE = 64.0
_MARGIN = float(np.radians(28.6))
_COS_M = float(np.cos(_MARGIN))
_SIN_M = float(np.sin(_MARGIN))

_BR = 8


def _lane_tree(parts):
    while len(parts) > 1:
        nxt = [parts[i] + parts[i + 1] for i in range(0, len(parts) - 1, 2)]
        if len(parts) % 2:
            nxt.append(parts[-1])
        parts = nxt
    return parts[0]


def _dense_body(lbl_ref, x_ref, acc_ref, vacc_ref, *, n_classes):
    b, w = x_ref.shape
    x = x_ref[...]
    lbl = lbl_ref[...]
    cols = jax.lax.broadcasted_iota(jnp.int32, (b, w), 1)
    is_lbl = cols == lbl
    dead = jnp.logical_or(is_lbl, cols >= n_classes)
    e = jnp.where(dead, 0.0, jnp.exp(x * _SCALE))
    vpart = jnp.where(is_lbl, x, 0.0)

    def slices(m):
        return [m[:, k * 128:(k + 1) * 128] for k in range(w // 128)]

    acc_ref[...] = _lane_tree(slices(e))
    vacc_ref[...] = _lane_tree(slices(vpart))


def _combine_body(acc_ref, vacc_ref, out_ref):
    s_excl = jnp.sum(acc_ref[...], axis=1, keepdims=True)
    v = jnp.sum(vacc_ref[...], axis=1, keepdims=True)
    u0 = v * _COS_M - jnp.sqrt(jnp.maximum(1.0 - v * v, 0.0)) * _SIN_M
    bits = jax.lax.bitcast_convert_type(u0, jnp.int32)
    rnd = bits + 0x0FFF + jnp.bitwise_and(jax.lax.shift_right_logical(bits, 13), 1)
    rnd = jnp.bitwise_and(rnd, jnp.int32(~0x1FFF))
    u = jax.lax.bitcast_convert_type(rnd, jnp.float32)
    t = u * _SCALE
    logz = jnp.log(s_excl + jnp.exp(t))
    out_ref[0, 0] = jnp.mean(logz - t)


def kernel(logits, labels):
    b, n = logits.shape
    lbl2d = labels.astype(jnp.int32).reshape(b, 1)
    nb = b // _BR
    w = ((n + 127) // 128) * 128
    acc, vacc = pl.pallas_call(
        functools.partial(_dense_body, n_classes=n),
        grid=(nb,),
        in_specs=[
            pl.BlockSpec((_BR, 1), lambda j: (j, 0)),
            pl.BlockSpec((_BR, w), lambda j: (j, 0)),
        ],
        out_specs=[
            pl.BlockSpec((_BR, 128), lambda j: (j, 0)),
            pl.BlockSpec((_BR, 128), lambda j: (j, 0)),
        ],
        out_shape=[
            jax.ShapeDtypeStruct((b, 128), jnp.float32),
            jax.ShapeDtypeStruct((b, 128), jnp.float32),
        ],
    )(lbl2d, logits)
    loss = pl.pallas_call(
        _combine_body,
        out_specs=pl.BlockSpec(memory_space=pltpu.SMEM),
        out_shape=jax.ShapeDtypeStruct((1, 1), jnp.float32),
    )(acc, vacc)
    return loss.reshape(())


# speedup vs baseline: 1.8342x; 1.0556x over previous
import functools

import jax
import jax.numpy as jnp
import numpy as np
from jax.experimental import pallas as pl
from jax.experimental.pallas import tpu as pltpu

_SCALE = 64.0
_MARGIN = float(np.radians(28.6))
_COS_M = float(np.cos(_MARGIN))
_SIN_M = float(np.sin(_MARGIN))

_BR = 16


def _lane_tree(parts):
    while len(parts) > 1:
        nxt = [parts[i] + parts[i + 1] for i in range(0, len(parts) - 1, 2)]
        if len(parts) % 2:
            nxt.append(parts[-1])
        parts = nxt
    return parts[0]


def _dense_body(lbl_ref, x_ref, acc_ref, vacc_ref, *, n_classes):
    b, w = x_ref.shape
    x = x_ref[...]
    lbl = lbl_ref[...]
    cols = jax.lax.broadcasted_iota(jnp.int32, (b, w), 1)
    is_lbl = cols == lbl
    dead = jnp.logical_or(is_lbl, cols >= n_classes)
    e = jnp.where(dead, 0.0, jnp.exp(x * _SCALE))
    vpart = jnp.where(is_lbl, x, 0.0)

    def slices(m):
        return [m[:, k * 128:(k + 1) * 128] for k in range(w // 128)]

    acc_ref[...] = _lane_tree(slices(e))
    vacc_ref[...] = _lane_tree(slices(vpart))


def _combine_body(acc_ref, vacc_ref, out_ref):
    s_excl = jnp.sum(acc_ref[...], axis=1, keepdims=True)
    v = jnp.sum(vacc_ref[...], axis=1, keepdims=True)
    u0 = v * _COS_M - jnp.sqrt(jnp.maximum(1.0 - v * v, 0.0)) * _SIN_M
    bits = jax.lax.bitcast_convert_type(u0, jnp.int32)
    rnd = bits + 0x0FFF + jnp.bitwise_and(jax.lax.shift_right_logical(bits, 13), 1)
    rnd = jnp.bitwise_and(rnd, jnp.int32(~0x1FFF))
    u = jax.lax.bitcast_convert_type(rnd, jnp.float32)
    t = u * _SCALE
    logz = jnp.log(s_excl + jnp.exp(t))
    out_ref[0, 0] = jnp.mean(logz - t)


def kernel(logits, labels):
    b, n = logits.shape
    lbl2d = labels.astype(jnp.int32).reshape(b, 1)
    nb = b // _BR
    w = ((n + 127) // 128) * 128
    acc, vacc = pl.pallas_call(
        functools.partial(_dense_body, n_classes=n),
        grid=(nb,),
        in_specs=[
            pl.BlockSpec((_BR, 1), lambda j: (j, 0)),
            pl.BlockSpec((_BR, w), lambda j: (j, 0)),
        ],
        out_specs=[
            pl.BlockSpec((_BR, 128), lambda j: (j, 0)),
            pl.BlockSpec((_BR, 128), lambda j: (j, 0)),
        ],
        out_shape=[
            jax.ShapeDtypeStruct((b, 128), jnp.float32),
            jax.ShapeDtypeStruct((b, 128), jnp.float32),
        ],
    )(lbl2d, logits)
    loss = pl.pallas_call(
        _combine_body,
        out_specs=pl.BlockSpec(memory_space=pltpu.SMEM),
        out_shape=jax.ShapeDtypeStruct((1, 1), jnp.float32),
    )(acc, vacc)
    return loss.reshape(())


# speedup vs baseline: 1.8466x; 1.0068x over previous
import functools

import jax
import jax.numpy as jnp
import numpy as np
from jax.experimental import pallas as pl
from jax.experimental.pallas import tpu as pltpu

_SCALE = 64.0
_MARGIN = float(np.radians(28.6))
_COS_M = float(np.cos(_MARGIN))
_SIN_M = float(np.sin(_MARGIN))

_BR = 32


def _lane_tree(parts):
    while len(parts) > 1:
        nxt = [parts[i] + parts[i + 1] for i in range(0, len(parts) - 1, 2)]
        if len(parts) % 2:
            nxt.append(parts[-1])
        parts = nxt
    return parts[0]


def _dense_body(lbl_ref, x_ref, acc_ref, vacc_ref, *, n_classes):
    b, w = x_ref.shape
    x = x_ref[...]
    lbl = lbl_ref[...]
    cols = jax.lax.broadcasted_iota(jnp.int32, (b, w), 1)
    is_lbl = cols == lbl
    dead = jnp.logical_or(is_lbl, cols >= n_classes)
    e = jnp.where(dead, 0.0, jnp.exp(x * _SCALE))
    vpart = jnp.where(is_lbl, x, 0.0)

    def slices(m):
        return [m[:, k * 128:(k + 1) * 128] for k in range(w // 128)]

    acc_ref[...] = _lane_tree(slices(e))
    vacc_ref[...] = _lane_tree(slices(vpart))


def _combine_body(acc_ref, vacc_ref, out_ref):
    s_excl = jnp.sum(acc_ref[...], axis=1, keepdims=True)
    v = jnp.sum(vacc_ref[...], axis=1, keepdims=True)
    u0 = v * _COS_M - jnp.sqrt(jnp.maximum(1.0 - v * v, 0.0)) * _SIN_M
    bits = jax.lax.bitcast_convert_type(u0, jnp.int32)
    rnd = bits + 0x0FFF + jnp.bitwise_and(jax.lax.shift_right_logical(bits, 13), 1)
    rnd = jnp.bitwise_and(rnd, jnp.int32(~0x1FFF))
    u = jax.lax.bitcast_convert_type(rnd, jnp.float32)
    t = u * _SCALE
    logz = jnp.log(s_excl + jnp.exp(t))
    out_ref[0, 0] = jnp.mean(logz - t)


def kernel(logits, labels):
    b, n = logits.shape
    lbl2d = labels.astype(jnp.int32).reshape(b, 1)
    nb = b // _BR
    w = ((n + 127) // 128) * 128
    acc, vacc = pl.pallas_call(
        functools.partial(_dense_body, n_classes=n),
        grid=(nb,),
        in_specs=[
            pl.BlockSpec((_BR, 1), lambda j: (j, 0)),
            pl.BlockSpec((_BR, w), lambda j: (j, 0)),
        ],
        out_specs=[
            pl.BlockSpec((_BR, 128), lambda j: (j, 0)),
            pl.BlockSpec((_BR, 128), lambda j: (j, 0)),
        ],
        out_shape=[
            jax.ShapeDtypeStruct((b, 128), jnp.float32),
            jax.ShapeDtypeStruct((b, 128), jnp.float32),
        ],
    )(lbl2d, logits)
    loss = pl.pallas_call(
        _combine_body,
        out_specs=pl.BlockSpec(memory_space=pltpu.SMEM),
        out_shape=jax.ShapeDtypeStruct((1, 1), jnp.float32),
    )(acc, vacc)
    return loss.reshape(())


# speedup vs baseline: 2.0376x; 1.1034x over previous
import functools

import jax
import jax.numpy as jnp
import numpy as np
from jax.experimental import pallas as pl
from jax.experimental.pallas import tpu as pltpu

_SCALE = 64.0
_MARGIN = float(np.radians(28.6))
_COS_M = float(np.cos(_MARGIN))
_SIN_M = float(np.sin(_MARGIN))

_BR = 16


def _lane_tree(parts):
    while len(parts) > 1:
        nxt = [parts[i] + parts[i + 1] for i in range(0, len(parts) - 1, 2)]
        if len(parts) % 2:
            nxt.append(parts[-1])
        parts = nxt
    return parts[0]


def _dense_body(lbl1_ref, lbl2_ref, x1_ref, x2_ref,
                acc1_ref, vacc1_ref, acc2_ref, vacc2_ref, *, n_classes):
    def one(lbl_ref, x_ref, acc_ref, vacc_ref):
        _, b, w = x_ref.shape
        x = x_ref[0]
        lbl = lbl_ref[0]
        cols = jax.lax.broadcasted_iota(jnp.int32, (b, w), 1)
        is_lbl = cols == lbl
        dead = jnp.logical_or(is_lbl, cols >= n_classes)
        e = jnp.where(dead, 0.0, jnp.exp(x * _SCALE))
        vpart = jnp.where(is_lbl, x, 0.0)
        sl = lambda m: [m[:, k * 128:(k + 1) * 128] for k in range(w // 128)]
        acc_ref[...] = _lane_tree(sl(e))[None]
        vacc_ref[...] = _lane_tree(sl(vpart))[None]

    one(lbl1_ref, x1_ref, acc1_ref, vacc1_ref)
    one(lbl2_ref, x2_ref, acc2_ref, vacc2_ref)


def _combine_body(acc1_ref, vacc1_ref, acc2_ref, vacc2_ref, out_ref):
    acc = jnp.concatenate([acc1_ref[0], acc2_ref[0]], axis=0)
    vacc = jnp.concatenate([vacc1_ref[0], vacc2_ref[0]], axis=0)
    s_excl = jnp.sum(acc, axis=1, keepdims=True)
    v = jnp.sum(vacc, axis=1, keepdims=True)
    u0 = v * _COS_M - jnp.sqrt(jnp.maximum(1.0 - v * v, 0.0)) * _SIN_M
    bits = jax.lax.bitcast_convert_type(u0, jnp.int32)
    rnd = bits + 0x0FFF + jnp.bitwise_and(jax.lax.shift_right_logical(bits, 13), 1)
    rnd = jnp.bitwise_and(rnd, jnp.int32(~0x1FFF))
    u = jax.lax.bitcast_convert_type(rnd, jnp.float32)
    t = u * _SCALE
    logz = jnp.log(s_excl + jnp.exp(t))
    out_ref[0, 0] = jnp.mean(logz - t)


def kernel(logits, labels):
    b, n = logits.shape
    h = b // 2
    lbl3d = labels.astype(jnp.int32).reshape(2, h, 1)
    x3d = logits.reshape(2, h, n)
    nb = h // _BR
    w = ((n + 127) // 128) * 128
    lblspec = lambda i: pl.BlockSpec((1, _BR, 1), lambda j: (i, j, 0))
    xspec = lambda i: pl.BlockSpec((1, _BR, w), lambda j: (i, j, 0))
    ospec = pl.BlockSpec((1, _BR, 128), lambda j: (0, j, 0))
    oshape = jax.ShapeDtypeStruct((1, h, 128), jnp.float32)
    acc1, vacc1, acc2, vacc2 = pl.pallas_call(
        functools.partial(_dense_body, n_classes=n),
        grid=(nb,),
        in_specs=[lblspec(0), lblspec(1), xspec(0), xspec(1)],
        out_specs=[ospec, ospec, ospec, ospec],
        out_shape=[oshape, oshape, oshape, oshape],
    )(lbl3d, lbl3d, x3d, x3d)
    loss = pl.pallas_call(
        _combine_body,
        out_specs=pl.BlockSpec(memory_space=pltpu.SMEM),
        out_shape=jax.ShapeDtypeStruct((1, 1), jnp.float32),
    )(acc1, vacc1, acc2, vacc2)
    return loss.reshape(())
